# baseline (speedup 1.0000x reference)
import functools
import math

import jax
import jax.numpy as jnp
from jax import lax
from jax.experimental import pallas as pl
from jax.experimental.pallas import tpu as pltpu
from jax.experimental.pallas import tpu_sc as plsc

_VOCAB = 1000000
_HIDDEN = 64
_B = 1024
_L = 200
_SCALE = math.sqrt(_HIDDEN)

_NC = 2
_NS = 16
_NW = _NC * _NS
_SEQ_PER_W = _B // _NW

_IA = 104
_IB = 96


def _derive_pairs(idx_ref, pair_ref, par_ref, n):
  offs = list(range(0, n - 15, 16))
  if offs[-1] + 16 < n:
    offs.append(n - 16)
  for off in offs:
    sl = pl.ds(off, 16)
    v = idx_ref[sl]
    pair_ref[sl] = v >> 1
    par_ref[sl] = (v & 1).astype(jnp.float32)


def _body(source_hbm, token_hbm, pos_hbm, out_hbm,
          pos_v, idx_a, idx_b, pair_a, pair_b, par_a, par_b,
          gbuf, obuf, sem):
  wid = lax.axis_index("s") * _NC + lax.axis_index("c")

  pltpu.sync_copy(pos_hbm.at[pl.ds(0, _L)], pos_v)

  def seq_body(s, carry):
    seq = wid * _SEQ_PER_W + s
    base = pl.multiple_of(seq * _L, 8)
    pltpu.sync_copy(source_hbm.at[pl.ds(base, _IA)], idx_a)
    pltpu.sync_copy(source_hbm.at[pl.ds(base + _IA, _IB)], idx_b)
    _derive_pairs(idx_a, pair_a, par_a, _IA)
    _derive_pairs(idx_b, pair_b, par_b, _IB)
    cp1 = pltpu.async_copy(token_hbm.at[pair_a], gbuf.at[pl.ds(0, _IA)], sem)
    cp2 = pltpu.async_copy(token_hbm.at[pair_b], gbuf.at[pl.ds(_IA, _IB)], sem)
    cp1.wait()
    cp2.wait()

    def make_row_body(par_ref, roff):
      def row_body(r, c2):
        rl = r - roff
        chunk = par_ref[pl.ds((rl // 16) * 16, 16)]
        lane = jnp.full((16, 1), rl % 16, jnp.int32)
        p = lax.gather(
            chunk, lane,
            dimension_numbers=lax.GatherDimensionNumbers(
                offset_dims=(), collapsed_slice_dims=(0,),
                start_index_map=(0,)),
            slice_sizes=(1,),
            mode=lax.GatherScatterMode.PROMISE_IN_BOUNDS)
        for c in range(_HIDDEN // 16):
          lo = gbuf[r, pl.ds(c * 16, 16)]
          hi = gbuf[r, pl.ds(_HIDDEN + c * 16, 16)]
          val = lo + p * (hi - lo)
          obuf[r, pl.ds(c * 16, 16)] = val * _SCALE + pos_v[r, pl.ds(c * 16, 16)]
        return c2

      return row_body

    lax.fori_loop(0, _IA, make_row_body(par_a, 0), 0)
    lax.fori_loop(_IA, _L, make_row_body(par_b, _IA), 0)
    pltpu.sync_copy(obuf, out_hbm.at[seq])
    return carry

  lax.fori_loop(0, _SEQ_PER_W, seq_body, 0)


@jax.jit
def kernel(source, token_table, pos_table):
  mesh = plsc.VectorSubcoreMesh(core_axis_name="c", subcore_axis_name="s",
                                num_cores=_NC, num_subcores=_NS)
  run = pl.kernel(
      _body,
      out_type=jax.ShapeDtypeStruct((_B, _L, _HIDDEN), jnp.float32),
      mesh=mesh,
      scratch_types=[
          pltpu.VMEM((_L, _HIDDEN), jnp.float32),
          pltpu.VMEM((_IA,), jnp.int32),
          pltpu.VMEM((_IB,), jnp.int32),
          pltpu.VMEM((_IA,), jnp.int32),
          pltpu.VMEM((_IB,), jnp.int32),
          pltpu.VMEM((_IA,), jnp.float32),
          pltpu.VMEM((_IB,), jnp.float32),
          pltpu.VMEM((_L, 2 * _HIDDEN), jnp.float32),
          pltpu.VMEM((_L, _HIDDEN), jnp.float32),
          pltpu.SemaphoreType.DMA,
      ],
  )
  return run(source.reshape(-1),
             token_table.reshape(_VOCAB // 2, 2 * _HIDDEN),
             pos_table)


# speedup vs baseline: 1.3130x; 1.3130x over previous
import math

import jax
import jax.numpy as jnp
from jax import lax
from jax.experimental import pallas as pl
from jax.experimental.pallas import tpu as pltpu
from jax.experimental.pallas import tpu_sc as plsc

_VOCAB = 1000000
_HIDDEN = 64
_B = 1024
_L = 200
_SCALE = math.sqrt(_HIDDEN)

_NC = 2
_NS = 16
_NW = _NC * _NS
_SEQ_PER_W = _B // _NW

_IA = 104
_IB = 96


def _body(source_hbm, token_hbm, pos_hbm, out_hbm,
          pos_v, idx_a, idx_b, tok, obuf, gsem, osem):
  wid = lax.axis_index("s") * _NC + lax.axis_index("c")

  pltpu.sync_copy(pos_hbm.at[pl.ds(0, _L)], pos_v)

  def start_gather(k, s):
    base = pl.multiple_of((wid * _SEQ_PER_W + s) * _L, 8)
    pltpu.sync_copy(source_hbm.at[pl.ds(base, _IA)], idx_a[k])
    pltpu.sync_copy(source_hbm.at[pl.ds(base + _IA, _IB)], idx_b[k])
    pltpu.async_copy(token_hbm.at[idx_a[k]], tok[k].at[pl.ds(0, _IA)], gsem[k])
    pltpu.async_copy(token_hbm.at[idx_b[k]], tok[k].at[pl.ds(_IA, _IB)], gsem[k])

  def wait_gather(k):
    pltpu.make_async_copy(token_hbm.at[idx_a[k]], tok[k].at[pl.ds(0, _IA)],
                          gsem[k]).wait()
    pltpu.make_async_copy(token_hbm.at[idx_b[k]], tok[k].at[pl.ds(_IA, _IB)],
                          gsem[k]).wait()

  def compute(k):
    tk, ob = tok[k], obuf[k]

    def row_body(r2, carry):
      for dr in range(2):
        r = r2 * 2 + dr
        for c in range(_HIDDEN // 16):
          sl = pl.ds(c * 16, 16)
          ob[r, sl] = tk[r, sl] * _SCALE + pos_v[r, sl]
      return carry

    lax.fori_loop(0, _L // 2, row_body, 0)

  def start_out(k, s):
    pltpu.async_copy(obuf[k], out_hbm.at[wid * _SEQ_PER_W + s], osem[k])

  def wait_out(k, s):
    pltpu.make_async_copy(obuf[k], out_hbm.at[wid * _SEQ_PER_W + s],
                          osem[k]).wait()

  start_gather(0, 0)
  start_gather(1, 1)
  wait_gather(0)
  compute(0)
  start_out(0, 0)
  start_gather(0, 2)
  wait_gather(1)
  compute(1)
  start_out(1, 1)
  start_gather(1, 3)

  def pair_body(i, carry):
    for k in range(2):
      s = 2 * i + k
      wait_gather(k)
      wait_out(k, s - 2)
      compute(k)
      start_out(k, s)

      @pl.when(i < _SEQ_PER_W // 2 - 1)
      def _():
        start_gather(k, s + 2)

    return carry

  lax.fori_loop(1, _SEQ_PER_W // 2, pair_body, 0)
  wait_out(0, _SEQ_PER_W - 2)
  wait_out(1, _SEQ_PER_W - 1)


@jax.jit
def kernel(source, token_table, pos_table):
  mesh = plsc.VectorSubcoreMesh(core_axis_name="c", subcore_axis_name="s",
                                num_cores=_NC, num_subcores=_NS)
  run = pl.kernel(
      _body,
      out_type=jax.ShapeDtypeStruct((_B, _L, _HIDDEN), jnp.float32),
      mesh=mesh,
      scratch_types=[
          pltpu.VMEM((_L, _HIDDEN), jnp.float32),
          [pltpu.VMEM((_IA,), jnp.int32)] * 2,
          [pltpu.VMEM((_IB,), jnp.int32)] * 2,
          [pltpu.VMEM((_L, _HIDDEN), jnp.float32)] * 2,
          [pltpu.VMEM((_L, _HIDDEN), jnp.float32)] * 2,
          [pltpu.SemaphoreType.DMA] * 2,
          [pltpu.SemaphoreType.DMA] * 2,
      ],
      compiler_params=pltpu.CompilerParams(use_tc_tiling_on_sc=False),
  )
  return run(source.reshape(-1), token_table, pos_table)
